# Initial kernel scaffold; baseline (speedup 1.0000x reference)
import functools

import jax
import jax.numpy as jnp
from jax import lax
from jax.experimental import pallas as pl
from jax.experimental.pallas import tpu as pltpu

K = 64
N = 32768
B = 128
R = N // 128
BLK = 1024
STEPS = N // BLK
ROWS_PER_STEP = BLK // 128


def _decoder_kernel(x_ref, oracle_ref, out_ref, w_ref, acc_ref):
    i = pl.program_id(0)

    @pl.when(i == 0)
    def _prologue():
        x = x_ref[...]
        bits = x.view(jnp.int32)

        hi0 = jnp.max(bits)
        lo0 = jnp.int32(0)

        def body(_, carry):
            lo, hi = carry
            mid = lo + (hi - lo + 1) // 2
            cnt = jnp.sum((bits >= mid).astype(jnp.int32))
            take = cnt >= K
            return jnp.where(take, mid, lo), jnp.where(take, hi, mid - 1)

        t_bits, _ = lax.fori_loop(0, 31, body, (lo0, hi0))

        c_gt = jnp.sum((bits > t_bits).astype(jnp.int32))
        need = K - c_gt
        eq = (bits == t_bits)

        eqf = eq.astype(jnp.float32)
        row_cnt = jnp.sum(eqf, axis=1, keepdims=True)
        rr = lax.broadcasted_iota(jnp.int32, (R, R), 0)
        cc = lax.broadcasted_iota(jnp.int32, (R, R), 1)
        lt_rows = (cc < rr).astype(jnp.float32)
        excl_row = jnp.dot(lt_rows, row_cnt,
                           preferred_element_type=jnp.float32)
        c1 = lax.broadcasted_iota(jnp.int32, (128, 128), 0)
        c2 = lax.broadcasted_iota(jnp.int32, (128, 128), 1)
        lt_cols = (c1 < c2).astype(jnp.float32)
        excl_col = jnp.dot(eqf, lt_cols,
                           preferred_element_type=jnp.float32)
        rank = excl_row + excl_col
        sel = (bits > t_bits) | (eq & (rank < need.astype(jnp.float32)))

        w_ref[...] = jnp.where(sel, x, 0.0)
        acc_ref[...] = jnp.zeros_like(acc_ref)

    blk = oracle_ref[...]
    acc = acc_ref[...]
    for q in range(ROWS_PER_STEP):
        w_row = w_ref[pl.ds(i * ROWS_PER_STEP + q, 1), :]
        acc += blk[:, q * 128:(q + 1) * 128] * w_row
    acc_ref[...] = acc

    @pl.when(i == STEPS - 1)
    def _epilogue():
        sample = jnp.sum(acc_ref[...], axis=1, keepdims=True)
        logp = jnp.log(sample + 1e-10)
        out_ref[0, 0] = jnp.sum(logp) / B


def kernel(concat_output, oracle_prob, k):
    x2d = concat_output.reshape(R, 128)
    out = pl.pallas_call(
        _decoder_kernel,
        grid=(STEPS,),
        in_specs=[
            pl.BlockSpec((R, 128), lambda i: (0, 0)),
            pl.BlockSpec((B, BLK), lambda i: (0, i)),
        ],
        out_specs=pl.BlockSpec((1, 1), lambda i: (0, 0)),
        out_shape=jax.ShapeDtypeStruct((1, 1), jnp.float32),
        scratch_shapes=[
            pltpu.VMEM((R, 128), jnp.float32),
            pltpu.VMEM((B, 128), jnp.float32),
        ],
    )(x2d, oracle_prob)
    return out[0, 0]


# speedup vs baseline: 1.6161x; 1.6161x over previous
import functools

import jax
import jax.numpy as jnp
from jax import lax
from jax.experimental import pallas as pl
from jax.experimental.pallas import tpu as pltpu

K = 64
N = 32768
B = 128
R = N // 128
BLK = 1024
STEPS = N // BLK
ROWS_PER_STEP = BLK // 128


def _decoder_kernel(x_ref, oracle_ref, out_ref, w_ref, acc_ref):
    i = pl.program_id(0)

    @pl.when(i == 0)
    def _prologue():
        x = x_ref[...]
        bits = x.view(jnp.int32)

        hi0 = jnp.max(bits)
        lo0 = jnp.int32(0)

        def body(_, carry):
            lo, hi = carry
            mid = lo + (hi - lo + 1) // 2
            cnt = jnp.sum((bits >= mid).astype(jnp.int32))
            take = cnt >= K
            return jnp.where(take, mid, lo), jnp.where(take, hi, mid - 1)

        t_bits, _ = lax.fori_loop(0, 31, body, (lo0, hi0))

        c_gt = jnp.sum((bits > t_bits).astype(jnp.int32))
        need = K - c_gt
        eq = (bits == t_bits)

        eqf = eq.astype(jnp.float32)
        row_cnt = jnp.sum(eqf, axis=1, keepdims=True)
        rr = lax.broadcasted_iota(jnp.int32, (R, R), 0)
        cc = lax.broadcasted_iota(jnp.int32, (R, R), 1)
        lt_rows = (cc < rr).astype(jnp.float32)
        excl_row = jnp.dot(lt_rows, row_cnt,
                           preferred_element_type=jnp.float32)
        c1 = lax.broadcasted_iota(jnp.int32, (128, 128), 0)
        c2 = lax.broadcasted_iota(jnp.int32, (128, 128), 1)
        lt_cols = (c1 < c2).astype(jnp.float32)
        excl_col = jnp.dot(eqf, lt_cols,
                           preferred_element_type=jnp.float32)
        rank = excl_row + excl_col
        sel = (bits > t_bits) | (eq & (rank < need.astype(jnp.float32)))

        w_ref[...] = jnp.where(sel, x, 0.0)
        acc_ref[...] = jnp.zeros_like(acc_ref)

    blk = oracle_ref[...]
    acc = acc_ref[...]
    for q in range(ROWS_PER_STEP):
        w_row = w_ref[pl.ds(i * ROWS_PER_STEP + q, 1), :]
        acc += blk[:, q * 128:(q + 1) * 128] * w_row
    acc_ref[...] = acc

    @pl.when(i == STEPS - 1)
    def _epilogue():
        sample = jnp.sum(acc_ref[...], axis=1, keepdims=True)
        logp = jnp.log(sample + 1e-10)
        out_ref[...] = jnp.sum(logp, keepdims=True) / B


def kernel(concat_output, oracle_prob, k):
    x2d = concat_output.reshape(R, 128)
    out = pl.pallas_call(
        _decoder_kernel,
        grid=(STEPS,),
        in_specs=[
            pl.BlockSpec((R, 128), lambda i: (0, 0)),
            pl.BlockSpec((B, BLK), lambda i: (0, i)),
        ],
        out_specs=pl.BlockSpec((1, 1), lambda i: (0, 0)),
        out_shape=jax.ShapeDtypeStruct((1, 1), jnp.float32),
        scratch_shapes=[
            pltpu.VMEM((R, 128), jnp.float32),
            pltpu.VMEM((B, 128), jnp.float32),
        ],
    )(x2d, oracle_prob)
    return out[0, 0]
